# Initial kernel scaffold; baseline (speedup 1.0000x reference)
import jax
import jax.numpy as jnp
from jax.experimental import pallas as pl


def kernel(x, router_w, lora_a, lora_b):
    raise NotImplementedError("write your pallas kernel here")


# speedup vs baseline: 10.0411x; 10.0411x over previous
import jax
import jax.numpy as jnp
from jax.experimental import pallas as pl
from jax.experimental.pallas import tpu as pltpu

_B, _L, _H, _E, _R, _TOP_K = 4, 2048, 1024, 8, 64, 2
_ER = _E * _R


def _moe_body(cls_ref, rw_ref, a_ref, b_ref, x_ref, o_ref):
    b = pl.program_id(0)
    cls_row = cls_ref[pl.ds(b, 1), :]
    logits = jax.lax.dot_general(
        cls_row, rw_ref[...],
        (((1,), (1,)), ((), ())),
        preferred_element_type=jnp.float32,
        precision=jax.lax.Precision.HIGHEST,
    )
    eidx = jax.lax.broadcasted_iota(jnp.int32, (1, _E), 1)
    m1 = jnp.max(logits, axis=1, keepdims=True)
    i1 = jnp.min(jnp.where(logits == m1, eidx, _E), axis=1, keepdims=True)
    rest = jnp.where(eidx == i1, -jnp.inf, logits)
    m2 = jnp.max(rest, axis=1, keepdims=True)
    i2 = jnp.min(jnp.where(rest == m2, eidx, _E), axis=1, keepdims=True)
    g1 = 1.0 / (1.0 + jnp.exp(m2 - m1))
    g2 = 1.0 - g1
    gates = jnp.where(eidx == i1, g1, jnp.where(eidx == i2, g2, 0.0))
    srow = jax.lax.broadcasted_iota(jnp.int32, (_E, _ER), 0)
    scol = jax.lax.broadcasted_iota(jnp.int32, (_E, _ER), 1)
    sel = (scol // _R == srow).astype(jnp.float32)
    gv = jnp.dot(gates, sel, preferred_element_type=jnp.float32)

    xb = x_ref[0].astype(jnp.bfloat16)
    low = jnp.dot(xb, a_ref[...], preferred_element_type=jnp.float32)
    low = (low * gv).astype(jnp.bfloat16)
    up = jnp.dot(low, b_ref[...], preferred_element_type=jnp.float32)
    o_ref[0] = x_ref[0] + up


def kernel(x, router_w, lora_a, lora_b):
    cls = x[:, 0, :]
    a_all = lora_a.transpose(1, 0, 2).reshape(_H, _ER).astype(jnp.bfloat16)
    b_all = lora_b.reshape(_ER, _H).astype(jnp.bfloat16)
    return pl.pallas_call(
        _moe_body,
        grid=(_B,),
        in_specs=[
            pl.BlockSpec((_B, _H), lambda b: (0, 0)),
            pl.BlockSpec((_E, _H), lambda b: (0, 0)),
            pl.BlockSpec((_H, _ER), lambda b: (0, 0)),
            pl.BlockSpec((_ER, _H), lambda b: (0, 0)),
            pl.BlockSpec((1, _L, _H), lambda b: (b, 0, 0)),
        ],
        out_specs=pl.BlockSpec((1, _L, _H), lambda b: (b, 0, 0)),
        out_shape=jax.ShapeDtypeStruct((_B, _L, _H), jnp.float32),
    )(cls, router_w, a_all, b_all, x)


# speedup vs baseline: 10.3392x; 1.0297x over previous
import jax
import jax.numpy as jnp
from jax.experimental import pallas as pl
from jax.experimental.pallas import tpu as pltpu

_B, _L, _H, _E, _R, _TOP_K = 4, 2048, 1024, 8, 64, 2
_KR = _TOP_K * _R


def _router_body(cls_ref, rw_ref, idx_ref, gates_ref):
    logits = jax.lax.dot_general(
        cls_ref[...], rw_ref[...],
        (((1,), (1,)), ((), ())),
        preferred_element_type=jnp.float32,
        precision=jax.lax.Precision.HIGHEST,
    )
    eidx = jax.lax.broadcasted_iota(jnp.int32, (_B, _E), 1)
    m1 = jnp.max(logits, axis=1, keepdims=True)
    i1 = jnp.min(jnp.where(logits == m1, eidx, _E), axis=1, keepdims=True)
    rest = jnp.where(eidx == i1, -jnp.inf, logits)
    m2 = jnp.max(rest, axis=1, keepdims=True)
    i2 = jnp.min(jnp.where(rest == m2, eidx, _E), axis=1, keepdims=True)
    g1 = 1.0 / (1.0 + jnp.exp(m2 - m1))
    idx_ref[...] = jnp.concatenate([i1, i2], axis=1)
    gates_ref[...] = jnp.concatenate([g1, 1.0 - g1], axis=1)


def _mix_body(idx_ref, gates_ref, a0_ref, a1_ref, b0_ref, b1_ref,
              x_ref, o_ref):
    b = pl.program_id(0)
    srow = jax.lax.broadcasted_iota(jnp.int32, (_TOP_K, _KR), 0)
    scol = jax.lax.broadcasted_iota(jnp.int32, (_TOP_K, _KR), 1)
    sel = (scol // _R == srow).astype(jnp.float32)
    gv = jnp.dot(gates_ref[pl.ds(b, 1), :], sel,
                 preferred_element_type=jnp.float32)

    a2 = jnp.concatenate([a0_ref[0], a1_ref[0]], axis=1)
    bcat = jnp.concatenate([b0_ref[0], b1_ref[0]], axis=0)
    xb = x_ref[0].astype(jnp.bfloat16)
    low = jnp.dot(xb, a2, preferred_element_type=jnp.float32)
    low = (low * gv).astype(jnp.bfloat16)
    up = jnp.dot(low, bcat, preferred_element_type=jnp.float32)
    o_ref[0] = x_ref[0] + up


def kernel(x, router_w, lora_a, lora_b):
    cls = x[:, 0, :]
    idx, gates = pl.pallas_call(
        _router_body,
        in_specs=[pl.BlockSpec((_B, _H), lambda: (0, 0)),
                  pl.BlockSpec((_E, _H), lambda: (0, 0))],
        out_specs=[pl.BlockSpec((_B, _TOP_K), lambda: (0, 0)),
                   pl.BlockSpec((_B, _TOP_K), lambda: (0, 0))],
        out_shape=[jax.ShapeDtypeStruct((_B, _TOP_K), jnp.int32),
                   jax.ShapeDtypeStruct((_B, _TOP_K), jnp.float32)],
    )(cls, router_w)

    a16 = lora_a.astype(jnp.bfloat16)
    b16 = lora_b.astype(jnp.bfloat16)
    grid_spec = pltpu.PrefetchScalarGridSpec(
        num_scalar_prefetch=1,
        grid=(_B,),
        in_specs=[
            pl.BlockSpec((_B, _TOP_K), lambda b, i: (0, 0)),
            pl.BlockSpec((1, _H, _R), lambda b, i: (i[b, 0], 0, 0)),
            pl.BlockSpec((1, _H, _R), lambda b, i: (i[b, 1], 0, 0)),
            pl.BlockSpec((1, _R, _H), lambda b, i: (i[b, 0], 0, 0)),
            pl.BlockSpec((1, _R, _H), lambda b, i: (i[b, 1], 0, 0)),
            pl.BlockSpec((1, _L, _H), lambda b, i: (b, 0, 0)),
        ],
        out_specs=pl.BlockSpec((1, _L, _H), lambda b, i: (b, 0, 0)),
    )
    return pl.pallas_call(
        _mix_body,
        grid_spec=grid_spec,
        out_shape=jax.ShapeDtypeStruct((_B, _L, _H), jnp.float32),
    )(idx, gates, a16, a16, b16, b16, x)
